# Initial kernel scaffold; baseline (speedup 1.0000x reference)
import functools

import jax
import jax.numpy as jnp
from jax import lax
from jax.experimental import pallas as pl
from jax.experimental.pallas import tpu as pltpu
from jax.experimental.pallas import tpu_sc as plsc

B = 4096
T = 200
D = 64
N = B * T

NC = 2
NS = 16
NW = NC * NS

ROWS_PER_W = B // NW
CR = 4
CHUNK = CR * T
NCHUNK = ROWS_PER_W // CR


def _make_kernel():
    mesh = plsc.VectorSubcoreMesh(core_axis_name="c", subcore_axis_name="s")

    @functools.partial(
        pl.kernel,
        out_type=jax.ShapeDtypeStruct((N, D), jnp.float32),
        mesh=mesh,
        scratch_types=[
            pltpu.VMEM((CHUNK,), jnp.int32),
            pltpu.VMEM((CHUNK,), jnp.int32),
            pltpu.VMEM((CHUNK, D), jnp.float32),
            pltpu.SemaphoreType.DMA,
        ],
    )
    def embed_kernel(xt_hbm, yt_hbm, xtab_hbm, ytab_hbm, ptab_hbm, out_hbm,
                     xidx, yidx, buf, sem):
        wid = lax.axis_index("s") * NC + lax.axis_index("c")
        base_tok = wid * (ROWS_PER_W * T)

        def body(ci, _):
            tok0 = base_tok + ci * CHUNK
            pltpu.sync_copy(xt_hbm.at[pl.ds(tok0, CHUNK)], xidx)
            pltpu.sync_copy(yt_hbm.at[pl.ds(tok0, CHUNK)], yidx)
            for r in range(CR):
                pltpu.sync_copy(ptab_hbm, buf.at[pl.ds(r * T, T)])
            pltpu.async_copy(xtab_hbm.at[xidx], buf, sem, add=True).wait()
            pltpu.async_copy(ytab_hbm.at[yidx], buf, sem, add=True).wait()
            pltpu.sync_copy(buf, out_hbm.at[pl.ds(tok0, CHUNK)])
            return ()

        lax.fori_loop(0, NCHUNK, body, ())

    return embed_kernel


_kernel = _make_kernel()


@jax.jit
def kernel(x_tokens, y_tokens, x_table, y_table, pos_table):
    xt = x_tokens.reshape(N).astype(jnp.int32)
    yt = y_tokens.reshape(N).astype(jnp.int32)
    out = _kernel(xt, yt, x_table, y_table, pos_table)
    return out.reshape(B, T, D)


# speedup vs baseline: 6.2108x; 6.2108x over previous
import functools

import jax
import jax.numpy as jnp
from jax import lax
from jax.experimental import pallas as pl
from jax.experimental.pallas import tpu as pltpu
from jax.experimental.pallas import tpu_sc as plsc

B = 4096
T = 200
D = 64
N = B * T

NC = 2
NS = 16
NW = NC * NS

ROWS_PER_W = B // NW
CR = 4
CHUNK = CR * T
NCHUNK = ROWS_PER_W // CR


def _make_kernel():
    mesh = plsc.VectorSubcoreMesh(core_axis_name="c", subcore_axis_name="s")

    @functools.partial(
        pl.kernel,
        out_type=jax.ShapeDtypeStruct((N, D), jnp.float32),
        mesh=mesh,
        scratch_types=[
            pltpu.VMEM((CHUNK,), jnp.int32),
            pltpu.VMEM((CHUNK,), jnp.int32),
            pltpu.VMEM((CHUNK, D), jnp.float32),
            pltpu.SemaphoreType.DMA,
        ],
        compiler_params=pltpu.CompilerParams(use_tc_tiling_on_sc=False),
    )
    def embed_kernel(xt_hbm, yt_hbm, xtab_hbm, ytab_hbm, ptab_hbm, out_hbm,
                     xidx, yidx, buf, sem):
        wid = lax.axis_index("s") * NC + lax.axis_index("c")
        base_tok = wid * (ROWS_PER_W * T)

        def body(ci, _):
            tok0 = base_tok + ci * CHUNK
            pltpu.sync_copy(xt_hbm.at[pl.ds(tok0, CHUNK)], xidx)
            pltpu.sync_copy(yt_hbm.at[pl.ds(tok0, CHUNK)], yidx)
            for r in range(CR):
                pltpu.sync_copy(ptab_hbm, buf.at[pl.ds(r * T, T)])
            pltpu.async_copy(xtab_hbm.at[xidx], buf, sem, add=True).wait()
            pltpu.async_copy(ytab_hbm.at[yidx], buf, sem, add=True).wait()
            pltpu.sync_copy(buf, out_hbm.at[pl.ds(tok0, CHUNK)])
            return ()

        lax.fori_loop(0, NCHUNK, body, ())

    return embed_kernel


_kernel = _make_kernel()


@jax.jit
def kernel(x_tokens, y_tokens, x_table, y_table, pos_table):
    xt = x_tokens.reshape(N).astype(jnp.int32)
    yt = y_tokens.reshape(N).astype(jnp.int32)
    out = _kernel(xt, yt, x_table, y_table, pos_table)
    return out.reshape(B, T, D)
